# baseline (speedup 1.0000x reference)
import functools

import jax
import jax.numpy as jnp
from jax.experimental import pallas as pl
from jax.experimental.pallas import tpu as pltpu


def _fpn_kernel_body(x_ref, s_ref, w_ref, u_ref, b_ref, o_ref, *, thl):
    w = w_ref[...]
    u = u_ref[...]
    b = b_ref[...]
    s_t = jnp.swapaxes(s_ref[0].astype(jnp.bfloat16), 0, 1)
    x3 = x_ref[0].astype(jnp.bfloat16)
    for hl in range(thl):
        xu = jax.lax.dot_general(
            x3[hl], u, (((0,), (0,)), ((), ())),
            preferred_element_type=jnp.float32)
        xu = xu + b
        for j in range(2):
            conv = jnp.dot(w, s_t[2 * hl + j],
                           preferred_element_type=jnp.float32)
            o_ref[0, :, 2 * hl + j, :] = (conv + xu).astype(o_ref.dtype)


def kernel(x_nchw, skip_nchw, weight, bias):
    N, Cp, H, W = x_nchw.shape
    _, Cs, Hs, Ws = skip_nchw.shape

    TH = 64
    THL = TH // 2
    grid = (N, Hs // TH)

    x_nhwc = jnp.transpose(x_nchw, (0, 2, 3, 1))

    w2 = weight.reshape(Cp, Cs).astype(jnp.bfloat16)
    u = jnp.repeat(jnp.eye(W, dtype=jnp.bfloat16), 2, axis=1)
    b2 = jnp.broadcast_to(bias.astype(jnp.float32)[:, None], (Cp, 2 * W))

    body = functools.partial(_fpn_kernel_body, thl=THL)

    out = pl.pallas_call(
        body,
        out_shape=jax.ShapeDtypeStruct((N, Cp, Hs, Ws), x_nchw.dtype),
        grid=grid,
        in_specs=[
            pl.BlockSpec((1, THL, W, Cp), lambda n, t: (n, t, 0, 0)),
            pl.BlockSpec((1, Cs, TH, Ws), lambda n, t: (n, 0, t, 0)),
            pl.BlockSpec((Cp, Cs), lambda n, t: (0, 0)),
            pl.BlockSpec((W, 2 * W), lambda n, t: (0, 0)),
            pl.BlockSpec((Cp, 2 * W), lambda n, t: (0, 0)),
        ],
        out_specs=pl.BlockSpec((1, Cp, TH, Ws), lambda n, t: (n, 0, t, 0)),
        compiler_params=pltpu.CompilerParams(
            dimension_semantics=("parallel", "parallel"),
            vmem_limit_bytes=64 * 2**20,
        ),
    )(x_nhwc, skip_nchw, w2, u, b2)
    return out


# speedup vs baseline: 1.0159x; 1.0159x over previous
import functools

import jax
import jax.numpy as jnp
from jax.experimental import pallas as pl
from jax.experimental.pallas import tpu as pltpu


def _fpn_kernel_body(x_ref, s_ref, w_ref, b_ref, o_ref, *, thl):
    w = w_ref[...]
    b = b_ref[...]
    wlo = x_ref.shape[2]
    row = jax.lax.broadcasted_iota(jnp.int32, (wlo, 2 * wlo), 0)
    col = jax.lax.broadcasted_iota(jnp.int32, (wlo, 2 * wlo), 1)
    u = (col // 2 == row).astype(jnp.bfloat16)
    s_t = jnp.swapaxes(s_ref[0].astype(jnp.bfloat16), 0, 1)
    x3 = x_ref[0].astype(jnp.bfloat16)
    for hl in range(thl):
        xu = jax.lax.dot_general(
            x3[hl], u, (((0,), (0,)), ((), ())),
            preferred_element_type=jnp.float32)
        xu = xu + b
        for j in range(2):
            conv = jnp.dot(w, s_t[2 * hl + j],
                           preferred_element_type=jnp.float32)
            o_ref[0, :, 2 * hl + j, :] = (conv + xu).astype(o_ref.dtype)


def kernel(x_nchw, skip_nchw, weight, bias):
    N, Cp, H, W = x_nchw.shape
    _, Cs, Hs, Ws = skip_nchw.shape

    TH = 32
    THL = TH // 2
    grid = (N, Hs // TH)

    x_nhwc = jnp.transpose(x_nchw, (0, 2, 3, 1))

    w2 = weight.reshape(Cp, Cs).astype(jnp.bfloat16)
    b2 = jnp.broadcast_to(bias.astype(jnp.float32)[:, None], (Cp, 2 * W))

    body = functools.partial(_fpn_kernel_body, thl=THL)

    out = pl.pallas_call(
        body,
        out_shape=jax.ShapeDtypeStruct((N, Cp, Hs, Ws), x_nchw.dtype),
        grid=grid,
        in_specs=[
            pl.BlockSpec((1, THL, W, Cp), lambda n, t: (n, t, 0, 0)),
            pl.BlockSpec((1, Cs, TH, Ws), lambda n, t: (n, 0, t, 0)),
            pl.BlockSpec((Cp, Cs), lambda n, t: (0, 0)),
            pl.BlockSpec((Cp, 2 * W), lambda n, t: (0, 0)),
        ],
        out_specs=pl.BlockSpec((1, Cp, TH, Ws), lambda n, t: (n, 0, t, 0)),
        compiler_params=pltpu.CompilerParams(
            dimension_semantics=("parallel", "parallel"),
            vmem_limit_bytes=64 * 2**20,
        ),
    )(x_nhwc, skip_nchw, w2, b2)
    return out
